# baseline (speedup 1.0000x reference)
import jax
import jax.numpy as jnp
from jax.experimental import pallas as pl
from jax.experimental.pallas import tpu as pltpu

B, T, D = 16, 2048, 512
HALF = T
K = B
NB = 4


def _ring_body(seq_hbm, pe_hbm, out_hbm, pe_buf, in_bufs, out_bufs, pe_sem, in_sems, out_sems):
    def in_dma(i, slot):
        b, h = i, 0
        return pltpu.make_async_copy(
            seq_hbm.at[b, pl.ds(h * HALF, HALF)], in_bufs.at[slot], in_sems.at[slot]
        )

    def out_dma(i, slot):
        b, h = i, 0
        return pltpu.make_async_copy(
            out_bufs.at[slot], out_hbm.at[b, 0, pl.ds(h * HALF, HALF)], out_sems.at[slot]
        )

    pe_copy = pltpu.make_async_copy(pe_hbm.at[0], pe_buf, pe_sem)
    pe_copy.start()
    for i in range(NB):
        in_dma(i, i).start()
    pe_copy.wait()

    for i in range(K):
        slot = i % NB
        if i >= NB:
            out_dma(i - NB, slot).wait()
        in_dma(i, slot).wait()
        h = 0
        out_bufs[slot] = in_bufs[slot] * 2.0 + pe_buf[pl.ds(h * HALF, HALF)]
        out_dma(i, slot).start()
        if i + NB < K:
            in_dma(i + NB, slot).start()

    for i in range(K - NB, K):
        out_dma(i, i % NB).wait()


def kernel(seq, times, pe):
    del times
    out = pl.pallas_call(
        _ring_body,
        in_specs=[
            pl.BlockSpec(memory_space=pl.ANY),
            pl.BlockSpec(memory_space=pl.ANY),
        ],
        out_specs=pl.BlockSpec(memory_space=pl.ANY),
        out_shape=jax.ShapeDtypeStruct((B, 1, T, D), seq.dtype),
        scratch_shapes=[
            pltpu.VMEM((T, D), jnp.float32),
            pltpu.VMEM((NB, HALF, D), jnp.float32),
            pltpu.VMEM((NB, HALF, D), jnp.float32),
            pltpu.SemaphoreType.DMA,
            pltpu.SemaphoreType.DMA((NB,)),
            pltpu.SemaphoreType.DMA((NB,)),
        ],
    )(seq, pe)
    mask = jnp.ones((B, 1), dtype=bool)
    return (out, mask)


# speedup vs baseline: 1.0280x; 1.0280x over previous
import jax
import jax.numpy as jnp
from jax.experimental import pallas as pl
from jax.experimental.pallas import tpu as pltpu

B, T, D = 16, 2048, 512
CB = 2
K = B // CB
NB = 3


def _ring_body(seq_hbm, pe_hbm, out_hbm, pe_buf, in_bufs, out_bufs, pe_sem, in_sems, out_sems):
    def in_dma(i, slot):
        return pltpu.make_async_copy(
            seq_hbm.at[pl.ds(i * CB, CB)], in_bufs.at[slot], in_sems.at[slot]
        )

    def out_dma(i, slot):
        return pltpu.make_async_copy(
            out_bufs.at[slot], out_hbm.at[pl.ds(i * CB, CB)], out_sems.at[slot]
        )

    pe_copy = pltpu.make_async_copy(pe_hbm.at[0], pe_buf, pe_sem)
    pe_copy.start()
    for i in range(NB):
        in_dma(i, i).start()
    pe_copy.wait()

    for i in range(K):
        slot = i % NB
        if i >= NB:
            out_dma(i - NB, slot).wait()
        in_dma(i, slot).wait()
        out_bufs[slot, :, 0] = in_bufs[slot] * 2.0 + pe_buf[...]
        out_dma(i, slot).start()
        if i + NB < K:
            in_dma(i + NB, slot).start()

    for i in range(K - NB, K):
        out_dma(i, i % NB).wait()


def kernel(seq, times, pe):
    del times
    out = pl.pallas_call(
        _ring_body,
        in_specs=[
            pl.BlockSpec(memory_space=pl.ANY),
            pl.BlockSpec(memory_space=pl.ANY),
        ],
        out_specs=pl.BlockSpec(memory_space=pl.ANY),
        out_shape=jax.ShapeDtypeStruct((B, 1, T, D), seq.dtype),
        scratch_shapes=[
            pltpu.VMEM((T, D), jnp.float32),
            pltpu.VMEM((NB, CB, T, D), jnp.float32),
            pltpu.VMEM((NB, CB, 1, T, D), jnp.float32),
            pltpu.SemaphoreType.DMA,
            pltpu.SemaphoreType.DMA((NB,)),
            pltpu.SemaphoreType.DMA((NB,)),
        ],
    )(seq, pe)
    mask = jnp.ones((B, 1), dtype=bool)
    return (out, mask)
